# baseline (speedup 1.0000x reference)
import functools

import jax
import jax.numpy as jnp
from jax.experimental import pallas as pl
from jax.experimental.pallas import tpu as pltpu

_B, _F, _H, _W = 4, 96, 384, 384
_HW = _H * _W
_BL = 2048


def _pool_body(x_ref, key_ref, idx_ref):
    j = pl.program_id(1)
    xb = x_ref[0]
    val = jnp.max(xb, axis=0, keepdims=True)
    ci = jax.lax.broadcasted_iota(jnp.int32, xb.shape, 0)
    argc = jnp.min(jnp.where(xb == val, ci, _F), axis=0, keepdims=True)
    vbits = jax.lax.bitcast_convert_type(val, jnp.int32)
    sgn = jax.lax.shift_right_arithmetic(vbits, 31)
    key = vbits ^ (sgn | jnp.int32(-(2**31)))
    pos = jax.lax.broadcasted_iota(jnp.int32, (1, _BL), 1) + j * _BL
    key_ref[0] = key
    idx_ref[0] = argc * _HW + pos


def _pool(x):
    x3 = x.reshape(_B, _F, _HW)
    nj = _HW // _BL
    key, idx = pl.pallas_call(
        _pool_body,
        grid=(_B, nj),
        in_specs=[pl.BlockSpec((1, _F, _BL), lambda b, j: (b, 0, j))],
        out_specs=[
            pl.BlockSpec((1, 1, _BL), lambda b, j: (b * nj + j, 0, 0)),
            pl.BlockSpec((1, 1, _BL), lambda b, j: (b * nj + j, 0, 0)),
        ],
        out_shape=[
            jax.ShapeDtypeStruct((_B * nj, 1, _BL), jnp.int32),
            jax.ShapeDtypeStruct((_B * nj, 1, _BL), jnp.int32),
        ],
    )(x3)
    return key.reshape(_B, _HW), idx.reshape(_B, _HW)


def _decode(key, idx):
    ku = key.view(jnp.uint32)
    pos_sign = (ku >> 31).astype(jnp.uint32)
    vbits = jnp.where(pos_sign == 1, ku ^ jnp.uint32(0x80000000), ~ku)
    sv = vbits.view(jnp.float32)
    p = idx % _HW
    sh = (p // _W).astype(jnp.float32)
    sw = (p % _W).astype(jnp.float32)
    return jnp.stack([sv, idx.astype(jnp.float32), sh, sw], axis=-1)


def kernel(x):
    key, idx = _pool(x)
    order = jnp.argsort(key.view(jnp.uint32), axis=1, stable=True)
    skey = jnp.take_along_axis(key, order, axis=1)
    sidx = jnp.take_along_axis(idx, order, axis=1)
    return _decode(skey, sidx)
